# baseline (speedup 1.0000x reference)
import functools

import jax
import jax.numpy as jnp
from jax import lax
from jax.experimental import pallas as pl
from jax.experimental.pallas import tpu as pltpu
from jax.experimental.pallas import tpu_sc as plsc

_N = 50000
_E = 1600000
_F = 4
_ND = 32
_D = 128
_G = 128
_NLAYER = 2

_NS = 16
_EROWS = 12800
_EPAD = _EROWS * 128
_RPT = _EROWS // _NS
_CH = 8
_NSLOT = 4
_NOUT = _RPT // _CH
_NPAD = _N
_NSTRIPE = _NPAD // _NS
_ZCH = 125


def _sc_phase(table, f, att3, src2, dst2, out_h, sbA, dbA, abA, sbB, dbB,
              abB, rbuf, wbuf, aggr, gsems, ssems, stA, stB, s, row0, n0):
    def stage_start(g, sb, db, ab, sem):
        base = row0 + jnp.minimum(g * _CH, _RPT - _CH)
        pltpu.async_copy(src2.at[pl.ds(base, _CH)], sb, sem)
        pltpu.async_copy(dst2.at[pl.ds(base, _CH)], db, sem)
        pltpu.async_copy(att3.at[f, pl.ds(base, _CH)], ab, sem)

    def stage_wait(sb, db, ab, sem):
        pltpu.make_async_copy(src2.at[pl.ds(row0, _CH)], sb, sem).wait()
        pltpu.make_async_copy(dst2.at[pl.ds(row0, _CH)], db, sem).wait()
        pltpu.make_async_copy(att3.at[0, pl.ds(row0, _CH)], ab, sem).wait()

    def _zb(i, carry):
        wbuf[0, i, pl.ds(0, 16)] = jnp.zeros((16,), jnp.float32)
        wbuf[0, i, pl.ds(16, 16)] = jnp.zeros((16,), jnp.float32)
        return carry
    lax.fori_loop(0, _ZCH, _zb, 0)

    def _zi(i, carry):
        pltpu.async_copy(wbuf.at[0, pl.ds(0, _ZCH)],
                         aggr.at[pl.ds(n0 + i * _ZCH, _ZCH)], ssems[0])
        return carry
    lax.fori_loop(0, _NSTRIPE // _ZCH, _zi, 0)

    def _zw(i, carry):
        pltpu.make_async_copy(wbuf.at[0, pl.ds(0, _ZCH)],
                              aggr.at[pl.ds(n0, _ZCH)], ssems[0]).wait()
        return carry
    lax.fori_loop(0, _NSTRIPE // _ZCH, _zw, 0)
    plsc.subcore_barrier()

    def scale(slot, ab, j):
        def _scale(q, c2):
            offs = jnp.minimum(q * 4, 112)
            a16 = ab[j, pl.ds(offs, 16)]
            ubase = q * 4 - offs
            for u in range(4):
                e = q * 4 + u
                idx = jnp.full((16,), 0, jnp.int32) + (ubase + u)
                sp = a16.at[idx].get(mode="promise_in_bounds")
                w = rbuf[slot, e, :]
                lo = lax.bitcast_convert_type(lax.shift_left(w, 16),
                                              jnp.float32)
                hi = lax.bitcast_convert_type(w & jnp.int32(-65536),
                                              jnp.float32)
                wbuf[slot, e, pl.ds(0, 16)] = lo * sp
                wbuf[slot, e, pl.ds(16, 16)] = hi * sp
            return c2
        lax.fori_loop(0, 32, _scale, 0)

    def run_half(sb, db, ab, sb_o, db_o, ab_o, sem_other, g_next):
        gds = {}
        sds = {}
        for j in range(_NSLOT):
            gds[j] = pltpu.async_copy(table.at[sb.at[j]], rbuf.at[j],
                                      gsems[j])
        stage_start(g_next, sb_o, db_o, ab_o, sem_other)
        for j in range(_CH):
            slot = j % _NSLOT
            tgt = j + _NSLOT - 1
            if j >= 1 and tgt < _CH:
                sds[j - 1].wait()
                gds[tgt] = pltpu.async_copy(table.at[sb.at[tgt]],
                                            rbuf.at[tgt % _NSLOT],
                                            gsems[tgt % _NSLOT])
            gds[j].wait()
            scale(slot, ab, j)
            sds[j] = pltpu.async_copy(wbuf.at[slot], aggr.at[db.at[j]],
                                      ssems[slot], add=True)
        for j in range(_CH - _NSLOT, _CH):
            sds[j].wait()

    def _outer(i, carry):
        stage_wait(sbA, dbA, abA, stA)
        run_half(sbA, dbA, abA, sbB, dbB, abB, stB, 2 * i + 1)
        stage_wait(sbB, dbB, abB, stB)
        run_half(sbB, dbB, abB, sbA, dbA, abA, stA, 2 * i + 2)
        return carry

    stage_start(0, sbA, dbA, abA, stA)
    lax.fori_loop(0, _NOUT // 2, _outer, 0)
    stage_wait(sbA, dbA, abA, stA)
    plsc.subcore_barrier()
    pltpu.sync_copy(aggr.at[pl.ds(n0, _NSTRIPE)],
                    out_h.at[f, pl.ds(n0, _NSTRIPE)])
    plsc.subcore_barrier()


_sc_msgpass_cache = []


def _sc_msgpass(*args):
    if not _sc_msgpass_cache:
        @functools.partial(
            pl.kernel,
            out_type=jax.ShapeDtypeStruct((_F, _NPAD, _ND), jnp.float32),
            mesh=plsc.VectorSubcoreMesh(core_axis_name="c", subcore_axis_name="s"),
            scratch_types=(
                [pltpu.VMEM((_CH, 128), jnp.int32),
                 pltpu.VMEM((_CH, 128), jnp.int32),
                 pltpu.VMEM((_CH, 128), jnp.float32)] * 2 +
                [pltpu.VMEM((_NSLOT, 128, 16), jnp.int32),
                 pltpu.VMEM((_NSLOT, 128, _ND), jnp.float32),
                 pltpu.VMEM_SHARED((_NPAD, _ND), jnp.float32)] +
                [pltpu.SemaphoreType.DMA] * (2 * _NSLOT + 2)
            ),
            compiler_params=pltpu.CompilerParams(use_tc_tiling_on_sc=False),
        )
        def _body(t0, t1, t2, t3, src2, dst2, att3, out_h,
                  sbA, dbA, abA, sbB, dbB, abB, rbuf, wbuf, aggr, *sems):
            gsems = list(sems[:_NSLOT])
            ssems = list(sems[_NSLOT:2 * _NSLOT])
            stA, stB = sems[2 * _NSLOT], sems[2 * _NSLOT + 1]
            c = lax.axis_index("c")
            s = lax.axis_index("s")
            row0 = s * _RPT
            n0 = s * _NSTRIPE
            rest = (att3, src2, dst2, out_h, sbA, dbA, abA, sbB, dbB, abB,
                    rbuf, wbuf, aggr, gsems, ssems, stA, stB, s, row0, n0)

            @pl.when(c == 0)
            def _():
                _sc_phase(t0, 0, *rest)
                _sc_phase(t1, 1, *rest)

            @pl.when(c == 1)
            def _():
                _sc_phase(t2, 2, *rest)
                _sc_phase(t3, 3, *rest)

        _sc_msgpass_cache.append(_body)
    return _sc_msgpass_cache[0](*args)


_BN = 2000


def _split4(out, h4r):
    if h4r.dtype == jnp.int32:
        for f in range(_F):
            a = out[:, _ND * f:_ND * f + 16].astype(jnp.bfloat16)
            b = out[:, _ND * f + 16:_ND * (f + 1)].astype(jnp.bfloat16)
            au = lax.bitcast_convert_type(a, jnp.uint16).astype(jnp.uint32)
            bu = lax.bitcast_convert_type(b, jnp.uint16).astype(jnp.uint32)
            h4r[f] = lax.bitcast_convert_type(au | (bu << 16), jnp.int32)
    else:
        for f in range(_F):
            h4r[f] = out[:, _ND * f:_ND * (f + 1)]


def _lin_body(xr, wr, br, hr, h4r):
    out = jnp.dot(xr[...], wr[...], preferred_element_type=jnp.float32)
    out = out + br[...]
    hr[...] = out
    _split4(out, h4r)


_lin_call = pl.pallas_call(
    _lin_body,
    grid=(_N // _BN,),
    in_specs=[
        pl.BlockSpec((_BN, _D), lambda i: (i, 0)),
        pl.BlockSpec((_D, _D), lambda i: (0, 0)),
        pl.BlockSpec((1, _D), lambda i: (0, 0)),
    ],
    out_specs=[
        pl.BlockSpec((_BN, _D), lambda i: (i, 0)),
        pl.BlockSpec((_F, _BN, 16), lambda i: (0, i, 0)),
    ],
    out_shape=[
        jax.ShapeDtypeStruct((_N, _D), jnp.float32),
        jax.ShapeDtypeStruct((_F, _N, 16), jnp.int32),
    ],
)


def _sigmoid(x):
    return 1.0 / (1.0 + jnp.exp(-x))


def _gru_body(ar, pr, mr, br, hr, h4r):
    acat = jnp.concatenate([ar[f] for f in range(_F)], axis=1)
    prev = pr[...]

    def mm(v, k):
        return lax.dot_general(v, mr[k], (((1,), (0,)), ((), ())),
                               preferred_element_type=jnp.float32)

    conv = mm(acat, 0) + mm(prev, 1) + br[0]
    m = jnp.maximum(conv, 0.0)
    r = _sigmoid(mm(m, 2) + br[1] + mm(prev, 5) + br[4])
    z = _sigmoid(mm(m, 3) + br[2] + mm(prev, 6) + br[5])
    n = jnp.tanh(mm(m, 4) + br[3] + r * (mm(prev, 7) + br[6]))
    h = (1.0 - z) * n + z * prev
    hr[...] = h
    _split4(h, h4r)


def _make_gru_call(packed):
    nd = 16 if packed else _ND
    dt = jnp.int32 if packed else jnp.float32
    return pl.pallas_call(
        _gru_body,
        grid=(_N // _BN,),
        in_specs=[
            pl.BlockSpec((_F, _BN, _ND), lambda i: (0, i, 0)),
            pl.BlockSpec((_BN, _D), lambda i: (i, 0)),
            pl.BlockSpec((8, _D, _D), lambda i: (0, 0, 0)),
            pl.BlockSpec((7, 1, _D), lambda i: (0, 0, 0)),
        ],
        out_specs=[
            pl.BlockSpec((_BN, _D), lambda i: (i, 0)),
            pl.BlockSpec((_F, _BN, nd), lambda i: (0, i, 0)),
        ],
        out_shape=[
            jax.ShapeDtypeStruct((_N, _D), jnp.float32),
            jax.ShapeDtypeStruct((_F, _N, nd), dt),
        ],
    )


_gru_calls = [_make_gru_call(True), _make_gru_call(False)]

_BP = 2000


def _pool_body(br_, hr_, outr, acc, cnt):
    i = pl.program_id(0)

    @pl.when(i == 0)
    def _():
        acc[...] = jnp.zeros_like(acc)
        cnt[...] = jnp.zeros_like(cnt)

    b = br_[0]
    gids = lax.broadcasted_iota(jnp.int32, (_G, _BP), 0)
    oh = (jnp.broadcast_to(b, (_G, _BP)) == gids).astype(jnp.float32)
    h = hr_[...]
    acc[...] += lax.dot_general(oh, h, (((1,), (0,)), ((), ())),
                                preferred_element_type=jnp.float32)
    cnt[...] += lax.dot_general(oh, jnp.ones((_BP, _D), jnp.float32),
                                (((1,), (0,)), ((), ())),
                                preferred_element_type=jnp.float32)

    @pl.when(i == _N // _BP - 1)
    def _():
        outr[...] = acc[...] / jnp.maximum(cnt[...], 1.0)


_pool_call = pl.pallas_call(
    _pool_body,
    grid=(_N // _BP,),
    in_specs=[
        pl.BlockSpec((1, 1, _BP), lambda i: (i, 0, 0)),
        pl.BlockSpec((_BP, _D), lambda i: (i, 0)),
    ],
    out_specs=pl.BlockSpec((_G, _D), lambda i: (0, 0)),
    out_shape=jax.ShapeDtypeStruct((_G, _D), jnp.float32),
    scratch_shapes=[
        pltpu.VMEM((_G, _D), jnp.float32),
        pltpu.VMEM((_G, _D), jnp.float32),
    ],
)


def _block_diag(ws):
    f, a, b = ws.shape
    out = jnp.zeros((f * a, f * b), ws.dtype)
    for i in range(f):
        out = out.at[i * a:(i + 1) * a, i * b:(i + 1) * b].set(ws[i])
    return out


def kernel(x, edge_index, batch, att, W_lin, b_lin, W_rel, b_rel, W_root,
           W_ih, W_hh, b_ih, b_hh):
    f32 = jnp.float32
    src = edge_index[0].astype(jnp.int32)
    dst = edge_index[1].astype(jnp.int32)
    pad = _EPAD - _E
    src2 = jnp.pad(src, (0, pad)).reshape(_EROWS, 128)
    dst2 = jnp.pad(dst, (0, pad)).reshape(_EROWS, 128)
    att3 = jnp.pad(att.astype(f32), ((0, 0), (0, pad))).reshape(_F, _EROWS, 128)

    wlT = W_lin.reshape(_D, _D).T
    bl = b_lin.reshape(1, _D)

    mats, biases = [], []
    w_ir, w_iz, w_in = W_ih[:, 0:32], W_ih[:, 32:64], W_ih[:, 64:96]
    w_hr, w_hz, w_hn = W_hh[:, 0:32], W_hh[:, 32:64], W_hh[:, 64:96]
    b_ir, b_iz, b_in = b_ih[:, 0:32], b_ih[:, 32:64], b_ih[:, 64:96]
    b_hr, b_hz, b_hn = b_hh[:, 0:32], b_hh[:, 32:64], b_hh[:, 64:96]
    gate_mats = [_block_diag(jnp.transpose(w, (0, 2, 1)))
                 for w in (w_ir, w_iz, w_in, w_hr, w_hz, w_hn)]
    gate_biases = [w.reshape(1, _D) for w in (b_ir, b_iz, b_in, b_hr, b_hz, b_hn)]
    for l in range(_NLAYER):
        m_rel = _block_diag(jnp.transpose(W_rel[:, l], (0, 2, 1)))
        m_root = _block_diag(jnp.transpose(W_root[:, l], (0, 2, 1)))
        mats.append(jnp.stack([m_rel, m_root] + gate_mats))
        biases.append(jnp.stack([b_rel[:, l].reshape(1, _D)] + gate_biases))

    h, h4 = _lin_call(x, wlT, bl)
    for l in range(_NLAYER):
        aggr4 = _sc_msgpass(h4[0], h4[1], h4[2], h4[3], src2, dst2, att3)
        h, h4 = _gru_calls[l](aggr4, h, mats[l], biases[l])

    batch3 = batch.astype(jnp.int32).reshape(_N // _BP, 1, _BP)
    pooled_cat = _pool_call(batch3, h)
    pooled = pooled_cat.reshape(_G, _F, _ND).transpose(1, 0, 2)
    return (pooled, h4)


# speedup vs baseline: 1.0434x; 1.0434x over previous
import functools

import jax
import jax.numpy as jnp
from jax import lax
from jax.experimental import pallas as pl
from jax.experimental.pallas import tpu as pltpu
from jax.experimental.pallas import tpu_sc as plsc

_N = 50000
_E = 1600000
_F = 4
_ND = 32
_D = 128
_G = 128
_NLAYER = 2

_NS = 16
_EROWS = 12800
_EPAD = _EROWS * 128
_RPT = _EROWS // _NS
_CH = 8
_NSLOT = 4
_NOUT = _RPT // _CH
_NPAD = _N
_NSTRIPE = _NPAD // _NS
_ZCH = 125


def _sc_phase(table, f, att3, src2, dst2, out_h, sbA, dbA, abA, sbB, dbB,
              abB, rbuf, wbuf, aggr, gsems, ssems, stA, stB, s, row0, n0):
    def stage_start(g, sb, db, ab, sem):
        base = row0 + jnp.minimum(g * _CH, _RPT - _CH)
        pltpu.async_copy(src2.at[pl.ds(base, _CH)], sb, sem)
        pltpu.async_copy(dst2.at[pl.ds(base, _CH)], db, sem)
        pltpu.async_copy(att3.at[f, pl.ds(base, _CH)], ab, sem)

    def stage_wait(sb, db, ab, sem):
        pltpu.make_async_copy(src2.at[pl.ds(row0, _CH)], sb, sem).wait()
        pltpu.make_async_copy(dst2.at[pl.ds(row0, _CH)], db, sem).wait()
        pltpu.make_async_copy(att3.at[0, pl.ds(row0, _CH)], ab, sem).wait()

    def _zb(i, carry):
        wbuf[0, i, pl.ds(0, 16)] = jnp.zeros((16,), jnp.float32)
        wbuf[0, i, pl.ds(16, 16)] = jnp.zeros((16,), jnp.float32)
        return carry
    lax.fori_loop(0, _ZCH, _zb, 0)

    def _zi(i, carry):
        pltpu.async_copy(wbuf.at[0, pl.ds(0, _ZCH)],
                         aggr.at[pl.ds(n0 + i * _ZCH, _ZCH)], ssems[0])
        return carry
    lax.fori_loop(0, _NSTRIPE // _ZCH, _zi, 0)

    def _zw(i, carry):
        pltpu.make_async_copy(wbuf.at[0, pl.ds(0, _ZCH)],
                              aggr.at[pl.ds(n0, _ZCH)], ssems[0]).wait()
        return carry
    lax.fori_loop(0, _NSTRIPE // _ZCH, _zw, 0)
    plsc.subcore_barrier()

    def scale(slot, ab, j):
        def _scale(q, c2):
            offs = jnp.minimum(q * 8, 112)
            a16 = ab[j, pl.ds(offs, 16)]
            ubase = q * 8 - offs
            for u in range(8):
                e = q * 8 + u
                idx = jnp.full((16,), 0, jnp.int32) + (ubase + u)
                sp = a16.at[idx].get(mode="promise_in_bounds")
                w = rbuf[slot, e, :]
                lo = lax.bitcast_convert_type(lax.shift_left(w, 16),
                                              jnp.float32)
                hi = lax.bitcast_convert_type(w & jnp.int32(-65536),
                                              jnp.float32)
                wbuf[slot, e, pl.ds(0, 16)] = lo * sp
                wbuf[slot, e, pl.ds(16, 16)] = hi * sp
            return c2
        lax.fori_loop(0, 16, _scale, 0)

    def run_half(sb, db, ab, sb_o, db_o, ab_o, sem_other, g_next):
        gds = {}
        sds = {}
        for j in range(_NSLOT):
            gds[j] = pltpu.async_copy(table.at[sb.at[j]], rbuf.at[j],
                                      gsems[j])
        stage_start(g_next, sb_o, db_o, ab_o, sem_other)
        for j in range(_CH):
            slot = j % _NSLOT
            tgt = j + _NSLOT - 1
            if j >= 1 and tgt < _CH:
                sds[j - 1].wait()
                gds[tgt] = pltpu.async_copy(table.at[sb.at[tgt]],
                                            rbuf.at[tgt % _NSLOT],
                                            gsems[tgt % _NSLOT])
            gds[j].wait()
            scale(slot, ab, j)
            sds[j] = pltpu.async_copy(wbuf.at[slot], aggr.at[db.at[j]],
                                      ssems[slot], add=True)
        for j in range(_CH - _NSLOT, _CH):
            sds[j].wait()

    def _outer(i, carry):
        stage_wait(sbA, dbA, abA, stA)
        run_half(sbA, dbA, abA, sbB, dbB, abB, stB, 2 * i + 1)
        stage_wait(sbB, dbB, abB, stB)
        run_half(sbB, dbB, abB, sbA, dbA, abA, stA, 2 * i + 2)
        return carry

    stage_start(0, sbA, dbA, abA, stA)
    lax.fori_loop(0, _NOUT // 2, _outer, 0)
    stage_wait(sbA, dbA, abA, stA)
    plsc.subcore_barrier()
    pltpu.sync_copy(aggr.at[pl.ds(n0, _NSTRIPE)],
                    out_h.at[f, pl.ds(n0, _NSTRIPE)])
    plsc.subcore_barrier()


_sc_msgpass_cache = []


def _sc_msgpass(*args):
    if not _sc_msgpass_cache:
        @functools.partial(
            pl.kernel,
            out_type=jax.ShapeDtypeStruct((_F, _NPAD, _ND), jnp.float32),
            mesh=plsc.VectorSubcoreMesh(core_axis_name="c", subcore_axis_name="s"),
            scratch_types=(
                [pltpu.VMEM((_CH, 128), jnp.int32),
                 pltpu.VMEM((_CH, 128), jnp.int32),
                 pltpu.VMEM((_CH, 128), jnp.float32)] * 2 +
                [pltpu.VMEM((_NSLOT, 128, 16), jnp.int32),
                 pltpu.VMEM((_NSLOT, 128, _ND), jnp.float32),
                 pltpu.VMEM_SHARED((_NPAD, _ND), jnp.float32)] +
                [pltpu.SemaphoreType.DMA] * (2 * _NSLOT + 2)
            ),
            compiler_params=pltpu.CompilerParams(use_tc_tiling_on_sc=False),
        )
        def _body(tab, src2, dst2, att3, out_h,
                  sbA, dbA, abA, sbB, dbB, abB, rbuf, wbuf, aggr, *sems):
            gsems = list(sems[:_NSLOT])
            ssems = list(sems[_NSLOT:2 * _NSLOT])
            stA, stB = sems[2 * _NSLOT], sems[2 * _NSLOT + 1]
            c = lax.axis_index("c")
            s = lax.axis_index("s")
            row0 = s * _RPT
            n0 = s * _NSTRIPE
            rest = (att3, src2, dst2, out_h, sbA, dbA, abA, sbB, dbB, abB,
                    rbuf, wbuf, aggr, gsems, ssems, stA, stB, s, row0, n0)

            for p in range(2):
                f = 2 * c + p
                _sc_phase(tab.at[f], f, *rest)

        _sc_msgpass_cache.append(_body)
    return _sc_msgpass_cache[0](*args)


_BN = 2000


def _split4(out, h4r):
    if h4r.dtype == jnp.int32:
        for f in range(_F):
            a = out[:, _ND * f:_ND * f + 16].astype(jnp.bfloat16)
            b = out[:, _ND * f + 16:_ND * (f + 1)].astype(jnp.bfloat16)
            au = lax.bitcast_convert_type(a, jnp.uint16).astype(jnp.uint32)
            bu = lax.bitcast_convert_type(b, jnp.uint16).astype(jnp.uint32)
            h4r[f] = lax.bitcast_convert_type(au | (bu << 16), jnp.int32)
    else:
        for f in range(_F):
            h4r[f] = out[:, _ND * f:_ND * (f + 1)]


def _lin_body(xr, wr, br, hr, h4r):
    out = jnp.dot(xr[...], wr[...], preferred_element_type=jnp.float32)
    out = out + br[...]
    hr[...] = out
    _split4(out, h4r)


_lin_call = pl.pallas_call(
    _lin_body,
    grid=(_N // _BN,),
    in_specs=[
        pl.BlockSpec((_BN, _D), lambda i: (i, 0)),
        pl.BlockSpec((_D, _D), lambda i: (0, 0)),
        pl.BlockSpec((1, _D), lambda i: (0, 0)),
    ],
    out_specs=[
        pl.BlockSpec((_BN, _D), lambda i: (i, 0)),
        pl.BlockSpec((_F, _BN, 16), lambda i: (0, i, 0)),
    ],
    out_shape=[
        jax.ShapeDtypeStruct((_N, _D), jnp.float32),
        jax.ShapeDtypeStruct((_F, _N, 16), jnp.int32),
    ],
)


def _sigmoid(x):
    return 1.0 / (1.0 + jnp.exp(-x))


def _gru_body(ar, pr, mr, br, hr, h4r):
    acat = jnp.concatenate([ar[f] for f in range(_F)], axis=1)
    prev = pr[...]

    def mm(v, k):
        return lax.dot_general(v, mr[k], (((1,), (0,)), ((), ())),
                               preferred_element_type=jnp.float32)

    conv = mm(acat, 0) + mm(prev, 1) + br[0]
    m = jnp.maximum(conv, 0.0)
    r = _sigmoid(mm(m, 2) + br[1] + mm(prev, 5) + br[4])
    z = _sigmoid(mm(m, 3) + br[2] + mm(prev, 6) + br[5])
    n = jnp.tanh(mm(m, 4) + br[3] + r * (mm(prev, 7) + br[6]))
    h = (1.0 - z) * n + z * prev
    hr[...] = h
    _split4(h, h4r)


def _make_gru_call(packed):
    nd = 16 if packed else _ND
    dt = jnp.int32 if packed else jnp.float32
    return pl.pallas_call(
        _gru_body,
        grid=(_N // _BN,),
        in_specs=[
            pl.BlockSpec((_F, _BN, _ND), lambda i: (0, i, 0)),
            pl.BlockSpec((_BN, _D), lambda i: (i, 0)),
            pl.BlockSpec((8, _D, _D), lambda i: (0, 0, 0)),
            pl.BlockSpec((7, 1, _D), lambda i: (0, 0, 0)),
        ],
        out_specs=[
            pl.BlockSpec((_BN, _D), lambda i: (i, 0)),
            pl.BlockSpec((_F, _BN, nd), lambda i: (0, i, 0)),
        ],
        out_shape=[
            jax.ShapeDtypeStruct((_N, _D), jnp.float32),
            jax.ShapeDtypeStruct((_F, _N, nd), dt),
        ],
    )


_gru_calls = [_make_gru_call(True), _make_gru_call(False)]

_BP = 2000


def _pool_body(br_, hr_, outr, acc, cnt):
    i = pl.program_id(0)

    @pl.when(i == 0)
    def _():
        acc[...] = jnp.zeros_like(acc)
        cnt[...] = jnp.zeros_like(cnt)

    b = br_[0]
    gids = lax.broadcasted_iota(jnp.int32, (_G, _BP), 0)
    oh = (jnp.broadcast_to(b, (_G, _BP)) == gids).astype(jnp.float32)
    h = hr_[...]
    acc[...] += lax.dot_general(oh, h, (((1,), (0,)), ((), ())),
                                preferred_element_type=jnp.float32)
    cnt[...] += lax.dot_general(oh, jnp.ones((_BP, _D), jnp.float32),
                                (((1,), (0,)), ((), ())),
                                preferred_element_type=jnp.float32)

    @pl.when(i == _N // _BP - 1)
    def _():
        outr[...] = acc[...] / jnp.maximum(cnt[...], 1.0)


_pool_call = pl.pallas_call(
    _pool_body,
    grid=(_N // _BP,),
    in_specs=[
        pl.BlockSpec((1, 1, _BP), lambda i: (i, 0, 0)),
        pl.BlockSpec((_BP, _D), lambda i: (i, 0)),
    ],
    out_specs=pl.BlockSpec((_G, _D), lambda i: (0, 0)),
    out_shape=jax.ShapeDtypeStruct((_G, _D), jnp.float32),
    scratch_shapes=[
        pltpu.VMEM((_G, _D), jnp.float32),
        pltpu.VMEM((_G, _D), jnp.float32),
    ],
)


def _block_diag(ws):
    f, a, b = ws.shape
    out = jnp.zeros((f * a, f * b), ws.dtype)
    for i in range(f):
        out = out.at[i * a:(i + 1) * a, i * b:(i + 1) * b].set(ws[i])
    return out


def kernel(x, edge_index, batch, att, W_lin, b_lin, W_rel, b_rel, W_root,
           W_ih, W_hh, b_ih, b_hh):
    f32 = jnp.float32
    src = edge_index[0].astype(jnp.int32)
    dst = edge_index[1].astype(jnp.int32)
    pad = _EPAD - _E
    src2 = jnp.pad(src, (0, pad)).reshape(_EROWS, 128)
    dst2 = jnp.pad(dst, (0, pad)).reshape(_EROWS, 128)
    att3 = jnp.pad(att.astype(f32), ((0, 0), (0, pad))).reshape(_F, _EROWS, 128)

    wlT = W_lin.reshape(_D, _D).T
    bl = b_lin.reshape(1, _D)

    mats, biases = [], []
    w_ir, w_iz, w_in = W_ih[:, 0:32], W_ih[:, 32:64], W_ih[:, 64:96]
    w_hr, w_hz, w_hn = W_hh[:, 0:32], W_hh[:, 32:64], W_hh[:, 64:96]
    b_ir, b_iz, b_in = b_ih[:, 0:32], b_ih[:, 32:64], b_ih[:, 64:96]
    b_hr, b_hz, b_hn = b_hh[:, 0:32], b_hh[:, 32:64], b_hh[:, 64:96]
    gate_mats = [_block_diag(jnp.transpose(w, (0, 2, 1)))
                 for w in (w_ir, w_iz, w_in, w_hr, w_hz, w_hn)]
    gate_biases = [w.reshape(1, _D) for w in (b_ir, b_iz, b_in, b_hr, b_hz, b_hn)]
    for l in range(_NLAYER):
        m_rel = _block_diag(jnp.transpose(W_rel[:, l], (0, 2, 1)))
        m_root = _block_diag(jnp.transpose(W_root[:, l], (0, 2, 1)))
        mats.append(jnp.stack([m_rel, m_root] + gate_mats))
        biases.append(jnp.stack([b_rel[:, l].reshape(1, _D)] + gate_biases))

    h, h4 = _lin_call(x, wlT, bl)
    for l in range(_NLAYER):
        aggr4 = _sc_msgpass(h4, src2, dst2, att3)
        h, h4 = _gru_calls[l](aggr4, h, mats[l], biases[l])

    batch3 = batch.astype(jnp.int32).reshape(_N // _BP, 1, _BP)
    pooled_cat = _pool_call(batch3, h)
    pooled = pooled_cat.reshape(_G, _F, _ND).transpose(1, 0, 2)
    return (pooled, h4)


# speedup vs baseline: 1.0719x; 1.0273x over previous
import functools

import jax
import jax.numpy as jnp
from jax import lax
from jax.experimental import pallas as pl
from jax.experimental.pallas import tpu as pltpu
from jax.experimental.pallas import tpu_sc as plsc

_N = 50000
_E = 1600000
_F = 4
_ND = 32
_D = 128
_G = 128
_NLAYER = 2

_NS = 16
_EROWS = 12800
_EPAD = _EROWS * 128
_RPT = _EROWS // _NS
_CH = 10
_NSLOT = 5
_NOUT = _RPT // _CH
_NPAD = 50176
_NSTRIPE = _NPAD // _NS
_ZCH = 112


def _sc_phase(table, f, att3, src2, dst2, out_h, sbA, dbA, abA, sbB, dbB,
              abB, rbuf, aggr, gsems, ssems, stA, stB, s, row0, n0):
    def stage_start(g, sb, db, ab, sem):
        base = row0 + jnp.minimum(g * _CH, _RPT - _CH)
        pltpu.async_copy(src2.at[pl.ds(base, _CH)], sb, sem)
        pltpu.async_copy(dst2.at[pl.ds(base, _CH)], db, sem)
        pltpu.async_copy(att3.at[f, pl.ds(base, _CH)], ab, sem)

    def stage_wait(sb, db, ab, sem):
        pltpu.make_async_copy(src2.at[pl.ds(row0, _CH)], sb, sem).wait()
        pltpu.make_async_copy(dst2.at[pl.ds(row0, _CH)], db, sem).wait()
        pltpu.make_async_copy(att3.at[0, pl.ds(row0, _CH)], ab, sem).wait()

    def _zb(i, carry):
        rbuf[0, i, pl.ds(0, 16)] = jnp.zeros((16,), jnp.float32)
        rbuf[0, i, pl.ds(16, 16)] = jnp.zeros((16,), jnp.float32)
        return carry
    lax.fori_loop(0, _ZCH, _zb, 0)
    zds = [pltpu.async_copy(rbuf.at[0, pl.ds(0, _ZCH)],
                            aggr.at[pl.ds(n0 + i * _ZCH, _ZCH)], ssems[0])
           for i in range(_NSTRIPE // _ZCH)]
    for d in zds:
        d.wait()
    plsc.subcore_barrier()

    def scale(slot, ab, j):
        def _scale(q, c2):
            offs = jnp.minimum(q * 8, 112)
            a16 = ab[j, pl.ds(offs, 16)]
            ubase = q * 8 - offs
            for u in range(8):
                e = q * 8 + u
                idx = jnp.full((16,), 0, jnp.int32) + (ubase + u)
                sp = a16.at[idx].get(mode="promise_in_bounds")
                rbuf[slot, e, pl.ds(0, 16)] = rbuf[slot, e, pl.ds(0, 16)] * sp
                rbuf[slot, e, pl.ds(16, 16)] = rbuf[slot, e, pl.ds(16, 16)] * sp
            return c2
        lax.fori_loop(0, 16, _scale, 0)

    def run_half(sb, db, ab, sb_o, db_o, ab_o, sem_other, g_next):
        gds = {}
        sds = {}
        for j in range(_NSLOT):
            gds[j] = pltpu.async_copy(table.at[sb.at[j]], rbuf.at[j],
                                      gsems[j])
        stage_start(g_next, sb_o, db_o, ab_o, sem_other)
        for j in range(_CH):
            slot = j % _NSLOT
            tgt = j + _NSLOT - 1
            if j >= 1 and tgt < _CH:
                sds[j - 1].wait()
                gds[tgt] = pltpu.async_copy(table.at[sb.at[tgt]],
                                            rbuf.at[tgt % _NSLOT],
                                            gsems[tgt % _NSLOT])
            gds[j].wait()
            scale(slot, ab, j)
            sds[j] = pltpu.async_copy(rbuf.at[slot], aggr.at[db.at[j]],
                                      ssems[slot], add=True)
        for j in range(_CH - _NSLOT, _CH):
            sds[j].wait()

    def _outer(i, carry):
        stage_wait(sbA, dbA, abA, stA)
        run_half(sbA, dbA, abA, sbB, dbB, abB, stB, 2 * i + 1)
        stage_wait(sbB, dbB, abB, stB)
        run_half(sbB, dbB, abB, sbA, dbA, abA, stA, 2 * i + 2)
        return carry

    stage_start(0, sbA, dbA, abA, stA)
    lax.fori_loop(0, _NOUT // 2, _outer, 0)
    stage_wait(sbA, dbA, abA, stA)
    plsc.subcore_barrier()
    pltpu.sync_copy(aggr.at[pl.ds(n0, _NSTRIPE)],
                    out_h.at[f, pl.ds(n0, _NSTRIPE)])
    plsc.subcore_barrier()


_sc_msgpass_cache = []


def _sc_msgpass(*args):
    if not _sc_msgpass_cache:
        @functools.partial(
            pl.kernel,
            out_type=jax.ShapeDtypeStruct((_F, _NPAD, _ND), jnp.float32),
            mesh=plsc.VectorSubcoreMesh(core_axis_name="c", subcore_axis_name="s"),
            scratch_types=(
                [pltpu.VMEM((_CH, 128), jnp.int32),
                 pltpu.VMEM((_CH, 128), jnp.int32),
                 pltpu.VMEM((_CH, 128), jnp.float32)] * 2 +
                [pltpu.VMEM((_NSLOT, 128, _ND), jnp.float32),
                 pltpu.VMEM_SHARED((_NPAD, _ND), jnp.float32)] +
                [pltpu.SemaphoreType.DMA] * (2 * _NSLOT + 2)
            ),
            compiler_params=pltpu.CompilerParams(use_tc_tiling_on_sc=False),
        )
        def _body(t0, t1, t2, t3, src2, dst2, att3, out_h,
                  sbA, dbA, abA, sbB, dbB, abB, rbuf, aggr, *sems):
            gsems = list(sems[:_NSLOT])
            ssems = list(sems[_NSLOT:2 * _NSLOT])
            stA, stB = sems[2 * _NSLOT], sems[2 * _NSLOT + 1]
            c = lax.axis_index("c")
            s = lax.axis_index("s")
            row0 = s * _RPT
            n0 = s * _NSTRIPE
            rest = (att3, src2, dst2, out_h, sbA, dbA, abA, sbB, dbB, abB,
                    rbuf, aggr, gsems, ssems, stA, stB, s, row0, n0)

            @pl.when(c == 0)
            def _():
                _sc_phase(t0, 0, *rest)
                _sc_phase(t1, 1, *rest)

            @pl.when(c == 1)
            def _():
                _sc_phase(t2, 2, *rest)
                _sc_phase(t3, 3, *rest)

        _sc_msgpass_cache.append(_body)
    return _sc_msgpass_cache[0](*args)


_BN = 2000


def _split4(out, h4r):
    for f in range(_F):
        h4r[f] = out[:, _ND * f:_ND * (f + 1)]


def _lin_body(xr, wr, br, hr, h4r):
    out = jnp.dot(xr[...], wr[...], preferred_element_type=jnp.float32)
    out = out + br[...]
    hr[...] = out
    _split4(out, h4r)


_lin_call = pl.pallas_call(
    _lin_body,
    grid=(_N // _BN,),
    in_specs=[
        pl.BlockSpec((_BN, _D), lambda i: (i, 0)),
        pl.BlockSpec((_D, _D), lambda i: (0, 0)),
        pl.BlockSpec((1, _D), lambda i: (0, 0)),
    ],
    out_specs=[
        pl.BlockSpec((_BN, _D), lambda i: (i, 0)),
        pl.BlockSpec((_F, _BN, _ND), lambda i: (0, i, 0)),
    ],
    out_shape=[
        jax.ShapeDtypeStruct((_N, _D), jnp.float32),
        jax.ShapeDtypeStruct((_F, _N, _ND), jnp.float32),
    ],
)


def _sigmoid(x):
    return 1.0 / (1.0 + jnp.exp(-x))


def _gru_body(ar, pr, mr, br, hr, h4r):
    acat = jnp.concatenate([ar[f] for f in range(_F)], axis=1)
    prev = pr[...]

    def mm(v, k):
        return lax.dot_general(v, mr[k], (((1,), (0,)), ((), ())),
                               preferred_element_type=jnp.float32)

    conv = mm(acat, 0) + mm(prev, 1) + br[0]
    m = jnp.maximum(conv, 0.0)
    r = _sigmoid(mm(m, 2) + br[1] + mm(prev, 5) + br[4])
    z = _sigmoid(mm(m, 3) + br[2] + mm(prev, 6) + br[5])
    n = jnp.tanh(mm(m, 4) + br[3] + r * (mm(prev, 7) + br[6]))
    h = (1.0 - z) * n + z * prev
    hr[...] = h
    _split4(h, h4r)


_gru_call = pl.pallas_call(
    _gru_body,
    grid=(_N // _BN,),
    in_specs=[
        pl.BlockSpec((_F, _BN, _ND), lambda i: (0, i, 0)),
        pl.BlockSpec((_BN, _D), lambda i: (i, 0)),
        pl.BlockSpec((8, _D, _D), lambda i: (0, 0, 0)),
        pl.BlockSpec((7, 1, _D), lambda i: (0, 0, 0)),
    ],
    out_specs=[
        pl.BlockSpec((_BN, _D), lambda i: (i, 0)),
        pl.BlockSpec((_F, _BN, _ND), lambda i: (0, i, 0)),
    ],
    out_shape=[
        jax.ShapeDtypeStruct((_N, _D), jnp.float32),
        jax.ShapeDtypeStruct((_F, _N, _ND), jnp.float32),
    ],
)

_BP = 2000


def _pool_body(br_, hr_, outr, acc, cnt):
    i = pl.program_id(0)

    @pl.when(i == 0)
    def _():
        acc[...] = jnp.zeros_like(acc)
        cnt[...] = jnp.zeros_like(cnt)

    b = br_[0]
    gids = lax.broadcasted_iota(jnp.int32, (_G, _BP), 0)
    oh = (jnp.broadcast_to(b, (_G, _BP)) == gids).astype(jnp.float32)
    h = hr_[...]
    acc[...] += lax.dot_general(oh, h, (((1,), (0,)), ((), ())),
                                preferred_element_type=jnp.float32)
    cnt[...] += lax.dot_general(oh, jnp.ones((_BP, _D), jnp.float32),
                                (((1,), (0,)), ((), ())),
                                preferred_element_type=jnp.float32)

    @pl.when(i == _N // _BP - 1)
    def _():
        outr[...] = acc[...] / jnp.maximum(cnt[...], 1.0)


_pool_call = pl.pallas_call(
    _pool_body,
    grid=(_N // _BP,),
    in_specs=[
        pl.BlockSpec((1, 1, _BP), lambda i: (i, 0, 0)),
        pl.BlockSpec((_BP, _D), lambda i: (i, 0)),
    ],
    out_specs=pl.BlockSpec((_G, _D), lambda i: (0, 0)),
    out_shape=jax.ShapeDtypeStruct((_G, _D), jnp.float32),
    scratch_shapes=[
        pltpu.VMEM((_G, _D), jnp.float32),
        pltpu.VMEM((_G, _D), jnp.float32),
    ],
)


def _block_diag(ws):
    f, a, b = ws.shape
    out = jnp.zeros((f * a, f * b), ws.dtype)
    for i in range(f):
        out = out.at[i * a:(i + 1) * a, i * b:(i + 1) * b].set(ws[i])
    return out


def kernel(x, edge_index, batch, att, W_lin, b_lin, W_rel, b_rel, W_root,
           W_ih, W_hh, b_ih, b_hh):
    f32 = jnp.float32
    src = edge_index[0].astype(jnp.int32)
    dst = edge_index[1].astype(jnp.int32)
    pad = _EPAD - _E
    src2 = jnp.pad(src, (0, pad)).reshape(_EROWS, 128)
    dst2 = jnp.pad(dst, (0, pad)).reshape(_EROWS, 128)
    att3 = jnp.pad(att.astype(f32), ((0, 0), (0, pad))).reshape(_F, _EROWS, 128)

    wlT = W_lin.reshape(_D, _D).T
    bl = b_lin.reshape(1, _D)

    mats, biases = [], []
    w_ir, w_iz, w_in = W_ih[:, 0:32], W_ih[:, 32:64], W_ih[:, 64:96]
    w_hr, w_hz, w_hn = W_hh[:, 0:32], W_hh[:, 32:64], W_hh[:, 64:96]
    b_ir, b_iz, b_in = b_ih[:, 0:32], b_ih[:, 32:64], b_ih[:, 64:96]
    b_hr, b_hz, b_hn = b_hh[:, 0:32], b_hh[:, 32:64], b_hh[:, 64:96]
    gate_mats = [_block_diag(jnp.transpose(w, (0, 2, 1)))
                 for w in (w_ir, w_iz, w_in, w_hr, w_hz, w_hn)]
    gate_biases = [w.reshape(1, _D) for w in (b_ir, b_iz, b_in, b_hr, b_hz, b_hn)]
    for l in range(_NLAYER):
        m_rel = _block_diag(jnp.transpose(W_rel[:, l], (0, 2, 1)))
        m_root = _block_diag(jnp.transpose(W_root[:, l], (0, 2, 1)))
        mats.append(jnp.stack([m_rel, m_root] + gate_mats))
        biases.append(jnp.stack([b_rel[:, l].reshape(1, _D)] + gate_biases))

    h, h4 = _lin_call(x, wlT, bl)
    for l in range(_NLAYER):
        aggr4 = _sc_msgpass(h4[0], h4[1], h4[2], h4[3], src2, dst2, att3)
        h, h4 = _gru_call(aggr4, h, mats[l], biases[l])

    batch3 = batch.astype(jnp.int32).reshape(_N // _BP, 1, _BP)
    pooled_cat = _pool_call(batch3, h)
    pooled = pooled_cat.reshape(_G, _F, _ND).transpose(1, 0, 2)
    return (pooled, h4)
